# baseline (speedup 1.0000x reference)
import functools

import jax
import jax.numpy as jnp
from jax import lax
from jax.experimental import pallas as pl
from jax.experimental.pallas import tpu as pltpu
from jax.experimental.pallas import tpu_sc as plsc

_R = 64
_K = 8
_BT = 512
_NW = 32
_NG = _BT // 16


def _mm1_body(x_ref, a_ref, yt_ref):
    y = jax.lax.dot_general(
        x_ref[...], a_ref[...], (((1,), (1,)), ((), ())),
        preferred_element_type=jnp.float32)
    yt_ref[0] = y.T


def _mm2_body(act_ref, b_ref, out_ref):
    act = act_ref[0].astype(jnp.bfloat16)
    out_ref[...] = jax.lax.dot_general(
        act, b_ref[...], (((0,), (1,)), ((), ())),
        preferred_element_type=jnp.float32)


def _sc_route(yt_hbm, d_hbm, act_hbm, y_v, d_v, act_v):
    wid = lax.axis_index("s") * 2 + lax.axis_index("c")
    pltpu.sync_copy(yt_hbm.at[wid], y_v)
    pltpu.sync_copy(d_hbm, d_v)

    def group(g, _):
        neg = jnp.full((16,), -jnp.inf, dtype=jnp.float32)
        t = (neg,) * _K

        def ins(e, t):
            t = list(t)
            v = jnp.abs(y_v[e, pl.ds(g * 16, 16)] + d_v[e, :])
            for k in range(_K):
                hi = jnp.maximum(t[k], v)
                v = jnp.minimum(t[k], v)
                t[k] = hi
            return tuple(t)

        t = lax.fori_loop(0, _R, ins, t, unroll=8)
        thr = t[_K - 1]

        def cnt_gt(e, c):
            a = jnp.abs(y_v[e, pl.ds(g * 16, 16)] + d_v[e, :])
            return c + jnp.where(a > thr, 1, 0).astype(jnp.int32)

        ngt = lax.fori_loop(0, _R, cnt_gt, jnp.zeros((16,), jnp.int32),
                            unroll=8)
        quota = jnp.full((16,), _K, jnp.int32) - ngt

        def emit(e, tc):
            yv = y_v[e, pl.ds(g * 16, 16)]
            a = jnp.abs(yv + d_v[e, :])
            tie = jnp.logical_and(a == thr, tc < quota)
            tc = tc + jnp.where(tie, 1, 0).astype(jnp.int32)
            sel = jnp.logical_or(a > thr, tie)
            act_v[e, pl.ds(g * 16, 16)] = jnp.where(sel, yv + yv, 0.0)
            return tc

        lax.fori_loop(0, _R, emit, jnp.zeros((16,), jnp.int32), unroll=8)
        return 0

    lax.fori_loop(0, _NG, group, 0)
    pltpu.sync_copy(act_v, act_hbm.at[wid])


@jax.jit
def kernel(x, A, B, d):
    n, in_f = x.shape
    out_f = B.shape[0]
    grid = (n // _BT,)

    yt = pl.pallas_call(
        _mm1_body,
        grid=grid,
        in_specs=[
            pl.BlockSpec((_BT, in_f), lambda i: (i, 0)),
            pl.BlockSpec((_R, in_f), lambda i: (0, 0)),
        ],
        out_specs=pl.BlockSpec((1, _R, _BT), lambda i: (i, 0, 0)),
        out_shape=jax.ShapeDtypeStruct((_NW, _R, _BT), jnp.float32),
        compiler_params=pltpu.CompilerParams(
            dimension_semantics=("parallel",)),
    )(x, A)

    d16 = jnp.tile(d.reshape(_R, 1), (1, 16))

    route = functools.partial(
        pl.kernel,
        mesh=plsc.VectorSubcoreMesh(core_axis_name="c", subcore_axis_name="s"),
        out_type=jax.ShapeDtypeStruct((_NW, _R, _BT), jnp.float32),
        scratch_types=[
            pltpu.VMEM((_R, _BT), jnp.float32),
            pltpu.VMEM((_R, 16), jnp.float32),
            pltpu.VMEM((_R, _BT), jnp.float32),
        ],
    )(_sc_route)
    act = route(yt, d16)

    out = pl.pallas_call(
        _mm2_body,
        grid=grid,
        in_specs=[
            pl.BlockSpec((1, _R, _BT), lambda i: (i, 0, 0)),
            pl.BlockSpec((out_f, _R), lambda i: (0, 0)),
        ],
        out_specs=pl.BlockSpec((_BT, out_f), lambda i: (i, 0)),
        out_shape=jax.ShapeDtypeStruct((n, out_f), jnp.float32),
        compiler_params=pltpu.CompilerParams(
            dimension_semantics=("parallel",)),
    )(act, B.astype(jnp.bfloat16))
    return out


# speedup vs baseline: 1.0328x; 1.0328x over previous
import functools

import jax
import jax.numpy as jnp
from jax import lax
from jax.experimental import pallas as pl
from jax.experimental.pallas import tpu as pltpu
from jax.experimental.pallas import tpu_sc as plsc

_R = 64
_K = 8
_BT = 512
_NW = 32
_BS = 256
_NG = _BS // 16
_HALF_BLOCKS = 16


def _mm1_body(x_ref, a_ref, yt_ref):
    y = jax.lax.dot_general(
        x_ref[...], a_ref[...], (((1,), (1,)), ((), ())),
        preferred_element_type=jnp.float32)
    yt_ref[0] = y.T


def _mm2_body(act1_ref, act2_ref, b_ref, out_ref):
    i = pl.program_id(0)
    act_t = jnp.where(i < _HALF_BLOCKS, act1_ref[0], act2_ref[0])
    act = act_t.astype(jnp.bfloat16)
    out_ref[...] = jax.lax.dot_general(
        act, b_ref[...], (((0,), (1,)), ((), ())),
        preferred_element_type=jnp.float32)


def _sc_route(yt_hbm, d_hbm, act_hbm, y_v, d_v, act_v):
    wid = lax.axis_index("s") * 2 + lax.axis_index("c")
    blk = wid // 2
    off = (wid % 2) * _BS
    pltpu.sync_copy(yt_hbm.at[blk, :, pl.ds(off, _BS)], y_v)
    pltpu.sync_copy(d_hbm, d_v)

    def group(g, _):
        neg = jnp.full((16,), -jnp.inf, dtype=jnp.float32)
        t = (neg,) * _K

        def ins(e, t):
            t = list(t)
            v = jnp.abs(y_v[e, pl.ds(g * 16, 16)] + d_v[e, :])
            for k in range(_K):
                hi = jnp.maximum(t[k], v)
                v = jnp.minimum(t[k], v)
                t[k] = hi
            return tuple(t)

        t = lax.fori_loop(0, _R, ins, t, unroll=8)
        thr = t[_K - 1]

        def cnt_gt(e, c):
            a = jnp.abs(y_v[e, pl.ds(g * 16, 16)] + d_v[e, :])
            return c + jnp.where(a > thr, 1, 0).astype(jnp.int32)

        ngt = lax.fori_loop(0, _R, cnt_gt, jnp.zeros((16,), jnp.int32),
                            unroll=8)
        quota = jnp.full((16,), _K, jnp.int32) - ngt

        def emit(e, tc):
            yv = y_v[e, pl.ds(g * 16, 16)]
            a = jnp.abs(yv + d_v[e, :])
            tie = jnp.logical_and(a == thr, tc < quota)
            tc = tc + jnp.where(tie, 1, 0).astype(jnp.int32)
            sel = jnp.logical_or(a > thr, tie)
            act_v[e, pl.ds(g * 16, 16)] = jnp.where(sel, yv + yv, 0.0)
            return tc

        lax.fori_loop(0, _R, emit, jnp.zeros((16,), jnp.int32), unroll=8)
        return 0

    lax.fori_loop(0, _NG, group, 0)
    pltpu.sync_copy(act_v, act_hbm.at[blk, :, pl.ds(off, _BS)])


def _mm1(x, A, half):
    n, in_f = x.shape
    off = half * _HALF_BLOCKS
    return pl.pallas_call(
        _mm1_body,
        grid=(_HALF_BLOCKS,),
        in_specs=[
            pl.BlockSpec((_BT, in_f), lambda i: (i + off, 0)),
            pl.BlockSpec((_R, in_f), lambda i: (0, 0)),
        ],
        out_specs=pl.BlockSpec((1, _R, _BT), lambda i: (i, 0, 0)),
        out_shape=jax.ShapeDtypeStruct((_HALF_BLOCKS, _R, _BT), jnp.float32),
        compiler_params=pltpu.CompilerParams(
            dimension_semantics=("parallel",)),
    )(x, A)


@jax.jit
def kernel(x, A, B, d):
    n, in_f = x.shape
    out_f = B.shape[0]

    d16 = jnp.tile(d.reshape(_R, 1), (1, 16))
    route = functools.partial(
        pl.kernel,
        mesh=plsc.VectorSubcoreMesh(core_axis_name="c", subcore_axis_name="s"),
        out_type=jax.ShapeDtypeStruct((_HALF_BLOCKS, _R, _BT), jnp.float32),
        scratch_types=[
            pltpu.VMEM((_R, _BS), jnp.float32),
            pltpu.VMEM((_R, 16), jnp.float32),
            pltpu.VMEM((_R, _BS), jnp.float32),
        ],
    )(_sc_route)

    yt1 = _mm1(x, A, 0)
    act1 = route(yt1, d16)
    yt2 = _mm1(x, A, 1)
    act2 = route(yt2, d16)

    out = pl.pallas_call(
        _mm2_body,
        grid=(n // _BT,),
        in_specs=[
            pl.BlockSpec((1, _R, _BT),
                         lambda i: (jnp.minimum(i, _HALF_BLOCKS - 1), 0, 0)),
            pl.BlockSpec((1, _R, _BT),
                         lambda i: (jnp.maximum(i - _HALF_BLOCKS, 0), 0, 0)),
            pl.BlockSpec((out_f, _R), lambda i: (0, 0)),
        ],
        out_specs=pl.BlockSpec((_BT, out_f), lambda i: (i, 0)),
        out_shape=jax.ShapeDtypeStruct((n, out_f), jnp.float32),
        compiler_params=pltpu.CompilerParams(
            dimension_semantics=("parallel",)),
    )(act1, act2, B.astype(jnp.bfloat16))
    return out


# speedup vs baseline: 1.1165x; 1.0810x over previous
import jax
import jax.numpy as jnp
from jax.experimental import pallas as pl
from jax.experimental.pallas import tpu as pltpu

_R = 64
_K = 8


def _body(x_ref, a_ref, b_ref, d_ref, out_ref):
    dn = (((1,), (1,)), ((), ()))
    y = jax.lax.dot_general(
        x_ref[...], a_ref[...], dn,
        preferred_element_type=jnp.float32)
    work = jnp.abs(y + d_ref[...])
    idx = jax.lax.broadcasted_iota(jnp.int32, work.shape, 1)
    for _ in range(_K):
        am = jnp.argmax(work, axis=1)
        work = jnp.where(idx == am[:, None], -jnp.inf, work)
    act = jnp.where(jnp.isneginf(work), y + y, 0.0).astype(jnp.bfloat16)
    out_ref[...] = jax.lax.dot_general(
        act, b_ref[...], dn, preferred_element_type=jnp.float32)


@jax.jit
def kernel(x, A, B, d):
    n, in_f = x.shape
    out_f = B.shape[0]
    bt = 512
    grid = (n // bt,)
    return pl.pallas_call(
        _body,
        grid=grid,
        in_specs=[
            pl.BlockSpec((bt, in_f), lambda i: (i, 0)),
            pl.BlockSpec((_R, in_f), lambda i: (0, 0)),
            pl.BlockSpec((out_f, _R), lambda i: (0, 0)),
            pl.BlockSpec((1, _R), lambda i: (0, 0)),
        ],
        out_specs=pl.BlockSpec((bt, out_f), lambda i: (i, 0)),
        out_shape=jax.ShapeDtypeStruct((n, out_f), jnp.float32),
        compiler_params=pltpu.CompilerParams(
            dimension_semantics=("parallel",)),
    )(x, A, B.astype(jnp.bfloat16), d.reshape(1, _R))
